# baseline (speedup 1.0000x reference)
import functools

import jax
import jax.numpy as jnp
from jax import lax
from jax.experimental import pallas as pl
from jax.experimental.pallas import tpu as pltpu
from jax.experimental.pallas import tpu_sc as plsc

N = 10000
NPAD = 10240
E = 320000
D = 128
NC = 2
NS = 16
NW = NC * NS
EPW = E // NW
K = 80
G = EPW // K
RPT = NPAD // NS
BS = 1024
LP = 56
SPW = BS // NW
ZERO_ROW = 10200
BR = 1024

_mesh = plsc.VectorSubcoreMesh(core_axis_name="c", subcore_axis_name="s",
                               num_cores=NC, num_subcores=NS)


def _wid():
    return lax.axis_index("s") * NC + lax.axis_index("c")


def _fill_rows(ref, nrows, width, value):
    vecs = width // 16
    val = jnp.full((16,), value, jnp.float32)

    def body(i, _):
        r = i // vecs
        c = i % vecs
        ref[r, pl.ds(c * 16, 16)] = val
        return 0

    lax.fori_loop(0, nrows * vecs, body, 0)


@functools.partial(
    pl.kernel,
    out_type=jax.ShapeDtypeStruct((NC, NPAD, D), jnp.float32),
    mesh=_mesh,
    scratch_types=[
        pltpu.VMEM((K,), jnp.int32),
        pltpu.VMEM((K, D), jnp.float32),
        pltpu.VMEM_SHARED((NPAD, D), jnp.float32),
    ],
)
def _deg_kernel(dst_hbm, degp_hbm, dbuf, ones_v, acc):
    cid = lax.axis_index("c")
    sid = lax.axis_index("s")
    wid = _wid()
    base = wid * EPW

    _fill_rows(ones_v, K, D, 0.0)
    for j in range(RPT // K):
        pltpu.sync_copy(ones_v, acc.at[pl.ds(sid * RPT + j * K, K)])
    plsc.subcore_barrier()
    _fill_rows(ones_v, K, D, 1.0)

    def body(g, _):
        pltpu.sync_copy(dst_hbm.at[pl.ds(base + g * K, K)], dbuf)
        pltpu.sync_copy(ones_v, acc.at[dbuf], add=True)
        return 0

    lax.fori_loop(0, G, body, 0)
    plsc.subcore_barrier()
    pltpu.sync_copy(acc.at[pl.ds(sid * RPT, RPT)],
                    degp_hbm.at[cid, pl.ds(sid * RPT, RPT)])


def _make_gather_scatter(nidx, nout, k):
    ipw = nidx // NW
    g = ipw // k
    rpt = nout // NS
    nz = min(k, rpt)

    @functools.partial(
        pl.kernel,
        out_type=jax.ShapeDtypeStruct((NC, nout, D), jnp.float32),
        mesh=_mesh,
        scratch_types=[
            pltpu.VMEM((k,), jnp.int32),
            pltpu.VMEM((k,), jnp.int32),
            pltpu.VMEM((k,), jnp.int32),
            pltpu.VMEM((k,), jnp.int32),
            pltpu.VMEM((k, D), jnp.float32),
            pltpu.VMEM((k, D), jnp.float32),
            pltpu.VMEM_SHARED((nout, D), jnp.float32),
            pltpu.SemaphoreType.DMA,
            pltpu.SemaphoreType.DMA,
        ],
    )
    def scatter_kernel(src_hbm, dst_hbm, u_hbm, accp_hbm,
                       sA, sB, dA, dB, rA, rB, acc, semA, semB):
        cid = lax.axis_index("c")
        sid = lax.axis_index("s")
        wid = _wid()
        base = wid * ipw

        _fill_rows(rA, nz, D, 0.0)
        for j in range(rpt // nz):
            pltpu.sync_copy(rA.at[pl.ds(0, nz)],
                            acc.at[pl.ds(sid * rpt + j * nz, nz)])
        plsc.subcore_barrier()

        def start(s, sbuf, rbuf, sem):
            pltpu.sync_copy(src_hbm.at[pl.ds(base + s * k, k)], sbuf)
            pltpu.async_copy(u_hbm.at[sbuf], rbuf, sem)

        def finish(s, sbuf, dbuf, rbuf, sem):
            pltpu.sync_copy(dst_hbm.at[pl.ds(base + s * k, k)], dbuf)
            pltpu.make_async_copy(u_hbm.at[sbuf], rbuf, sem).wait()
            pltpu.sync_copy(rbuf, acc.at[dbuf], add=True)

        start(0, sA, rA, semA)

        def body(i, _):
            gB = 2 * i + 1

            @pl.when(gB < g)
            def _():
                start(gB, sB, rB, semB)

            finish(2 * i, sA, dA, rA, semA)

            @pl.when(2 * i + 2 < g)
            def _():
                start(2 * i + 2, sA, rA, semA)

            @pl.when(gB < g)
            def _():
                finish(gB, sB, dB, rB, semB)

            return 0

        lax.fori_loop(0, (g + 1) // 2, body, 0)
        plsc.subcore_barrier()
        pltpu.sync_copy(acc.at[pl.ds(sid * rpt, rpt)],
                        accp_hbm.at[cid, pl.ds(sid * rpt, rpt)])

    return scatter_kernel


SI = BS * LP
_edge_kernel = _make_gather_scatter(E, NPAD, K)
_sent_kernel = _make_gather_scatter(SI, BS, 128)


def _dinv_block(degp, i):
    deg = 1.0 + jnp.sum(degp, axis=0)[:, :1]
    dinv = lax.rsqrt(deg)
    rid = i * BR + lax.broadcasted_iota(jnp.int32, (BR, 1), 0)
    return jnp.where(rid < N, dinv, 0.0)


def _tc1_body(degp_ref, x_ref, u_ref):
    dinv = _dinv_block(degp_ref[...], pl.program_id(0))
    u_ref[...] = x_ref[...] * dinv


def _tc2_body(degp_ref, accp_ref, u1_ref, w1_ref, b1_ref, w2_ref, u2_ref):
    dinv = _dinv_block(degp_ref[...], pl.program_id(0))
    px = (accp_ref[0] + accp_ref[1] + u1_ref[...]) * dinv
    h1 = jax.nn.relu(
        jnp.dot(px, w1_ref[...], preferred_element_type=jnp.float32)
        + b1_ref[...])
    g = jnp.dot(h1, w2_ref[...], preferred_element_type=jnp.float32)
    u2_ref[...] = g * dinv


def _tc3_body(degp_ref, accp_ref, u2_ref, h2_ref):
    dinv = _dinv_block(degp_ref[...], pl.program_id(0))
    h2_ref[...] = (accp_ref[0] + accp_ref[1] + u2_ref[...]) * dinv


def _tc4_body(sentp_ref, b2_ref, wf1_ref, bf1_ref, wf2_ref, bf2_ref,
              wf3_ref, bf3_ref, out_ref):
    s = sentp_ref[0] + sentp_ref[1] + 50.0 * b2_ref[...]
    o1 = jax.nn.relu(
        jnp.dot(s, wf1_ref[...], preferred_element_type=jnp.float32)
        + bf1_ref[...])
    o2 = jax.nn.relu(
        jnp.dot(o1, wf2_ref[...], preferred_element_type=jnp.float32)
        + bf2_ref[...])
    out_ref[...] = (
        jnp.dot(o2, wf3_ref[...], preferred_element_type=jnp.float32)
        + bf3_ref[...])


def _row_grid_call(body, in_specs_extra, out_width, *args):
    grid = (NPAD // BR,)
    degp_spec = pl.BlockSpec((NC, BR, D), lambda i: (0, i, 0))
    return pl.pallas_call(
        body,
        grid=grid,
        in_specs=[degp_spec] + in_specs_extra,
        out_specs=pl.BlockSpec((BR, out_width), lambda i: (i, 0)),
        out_shape=jax.ShapeDtypeStruct((NPAD, out_width), jnp.float32),
    )(*args)


_ROWS = pl.BlockSpec((BR, D), lambda i: (i, 0))
_ACCP = pl.BlockSpec((NC, BR, D), lambda i: (0, i, 0))


def _full(shape):
    return pl.BlockSpec(shape, lambda i: tuple(0 for _ in shape))


def kernel(sentence, x, edge_index, W1, b1, W2, b2,
           Wf1, bf1, Wf2, bf2, Wf3, bf3):
    f32 = jnp.float32
    src = edge_index[0]
    dst = edge_index[1]
    xp = jnp.zeros((NPAD, D), f32).at[:N].set(x)

    degp = _deg_kernel(dst)
    u1 = _row_grid_call(_tc1_body, [_ROWS], D, degp, xp)
    accp1 = _edge_kernel(src, dst, u1)
    u2 = _row_grid_call(
        _tc2_body,
        [_ACCP, _ROWS, _full((D, 256)), _full((1, 256)), _full((256, D))],
        D, degp, accp1, u1, W1, b1.reshape(1, -1), W2)
    accp2 = _edge_kernel(src, dst, u2)
    h2 = _row_grid_call(_tc3_body, [_ACCP, _ROWS], D,
                        degp, accp2, u2)

    sentp = jnp.pad(sentence, ((0, 0), (0, LP - sentence.shape[1])),
                    constant_values=ZERO_ROW).T.reshape(-1)
    sdst = jnp.tile(jnp.arange(BS, dtype=jnp.int32), LP)
    sent = _sent_kernel(sentp, sdst, h2)

    wf3p = jnp.zeros((D, 128), f32).at[:, :2].set(Wf3)
    bf3p = jnp.zeros((1, 128), f32).at[0, :2].set(bf3)
    outp = pl.pallas_call(
        _tc4_body,
        grid=(1,),
        in_specs=[_full((NC, BS, D)), _full((1, D)), _full((D, 256)),
                  _full((1, 256)), _full((256, D)), _full((1, D)),
                  _full((D, 128)), _full((1, 128))],
        out_specs=_full((BS, 128)),
        out_shape=jax.ShapeDtypeStruct((BS, 128), f32),
    )(sent, b2.reshape(1, -1), Wf1, bf1.reshape(1, -1), Wf2,
      bf2.reshape(1, -1), wf3p, bf3p)
    return outp[:, :2]


# speedup vs baseline: 1.0110x; 1.0110x over previous
import functools

import jax
import jax.numpy as jnp
from jax import lax
from jax.experimental import pallas as pl
from jax.experimental.pallas import tpu as pltpu
from jax.experimental.pallas import tpu_sc as plsc

N = 10000
NPAD = 10240
E = 320000
D = 128
NC = 2
NS = 16
NW = NC * NS
EPW = E // NW
K = 80
G = EPW // K
RPT = NPAD // NS
BS = 1024
LP = 56
SPW = BS // NW
ZERO_ROW = 10200
BR = 1024

_mesh = plsc.VectorSubcoreMesh(core_axis_name="c", subcore_axis_name="s",
                               num_cores=NC, num_subcores=NS)


def _wid():
    return lax.axis_index("s") * NC + lax.axis_index("c")


def _fill_rows(ref, nrows, width, value):
    vecs = width // 16
    val = jnp.full((16,), value, jnp.float32)

    def body(i, _):
        r = i // vecs
        c = i % vecs
        ref[r, pl.ds(c * 16, 16)] = val
        return 0

    lax.fori_loop(0, nrows * vecs, body, 0)


@functools.partial(
    pl.kernel,
    out_type=jax.ShapeDtypeStruct((NC, NPAD, D), jnp.float32),
    mesh=_mesh,
    scratch_types=[
        pltpu.VMEM((K,), jnp.int32),
        pltpu.VMEM((K, D), jnp.float32),
        pltpu.VMEM_SHARED((NPAD, D), jnp.float32),
    ],
)
def _deg_kernel(dst_hbm, degp_hbm, dbuf, ones_v, acc):
    cid = lax.axis_index("c")
    sid = lax.axis_index("s")
    wid = _wid()
    base = wid * EPW

    _fill_rows(ones_v, K, D, 0.0)
    for j in range(RPT // K):
        pltpu.sync_copy(ones_v, acc.at[pl.ds(sid * RPT + j * K, K)])
    plsc.subcore_barrier()
    _fill_rows(ones_v, K, D, 1.0)

    def body(g, _):
        pltpu.sync_copy(dst_hbm.at[pl.ds(base + g * K, K)], dbuf)
        pltpu.sync_copy(ones_v, acc.at[dbuf], add=True)
        return 0

    lax.fori_loop(0, G, body, 0)
    plsc.subcore_barrier()
    pltpu.sync_copy(acc.at[pl.ds(sid * RPT, RPT)],
                    degp_hbm.at[cid, pl.ds(sid * RPT, RPT)])


def _make_gather_scatter(nidx, nout, k):
    ipw = nidx // NW
    g = ipw // k
    rpt = nout // NS
    nz = min(k, rpt)

    @functools.partial(
        pl.kernel,
        out_type=jax.ShapeDtypeStruct((NC, nout, D), jnp.float32),
        mesh=_mesh,
        scratch_types=[
            pltpu.VMEM((k,), jnp.int32),
            pltpu.VMEM((k,), jnp.int32),
            pltpu.VMEM((k,), jnp.int32),
            pltpu.VMEM((k,), jnp.int32),
            pltpu.VMEM((k, D), jnp.float32),
            pltpu.VMEM((k, D), jnp.float32),
            pltpu.VMEM_SHARED((nout, D), jnp.float32),
            pltpu.SemaphoreType.DMA,
            pltpu.SemaphoreType.DMA,
        ],
    )
    def scatter_kernel(src_hbm, dst_hbm, u_hbm, accp_hbm,
                       sA, sB, dA, dB, rA, rB, acc, semA, semB):
        cid = lax.axis_index("c")
        sid = lax.axis_index("s")
        wid = _wid()
        base = wid * ipw

        _fill_rows(rA, nz, D, 0.0)
        for j in range(rpt // nz):
            pltpu.sync_copy(rA.at[pl.ds(0, nz)],
                            acc.at[pl.ds(sid * rpt + j * nz, nz)])
        plsc.subcore_barrier()

        def start(s, sbuf, rbuf, sem):
            pltpu.sync_copy(src_hbm.at[pl.ds(base + s * k, k)], sbuf)
            pltpu.async_copy(u_hbm.at[sbuf], rbuf, sem)

        def finish(s, sbuf, dbuf, rbuf, sem):
            pltpu.sync_copy(dst_hbm.at[pl.ds(base + s * k, k)], dbuf)
            pltpu.make_async_copy(u_hbm.at[sbuf], rbuf, sem).wait()
            pltpu.sync_copy(rbuf, acc.at[dbuf], add=True)

        start(0, sA, rA, semA)

        def body(i, _):
            gB = 2 * i + 1

            @pl.when(gB < g)
            def _():
                start(gB, sB, rB, semB)

            finish(2 * i, sA, dA, rA, semA)

            @pl.when(2 * i + 2 < g)
            def _():
                start(2 * i + 2, sA, rA, semA)

            @pl.when(gB < g)
            def _():
                finish(gB, sB, dB, rB, semB)

            return 0

        lax.fori_loop(0, (g + 1) // 2, body, 0)
        plsc.subcore_barrier()
        pltpu.sync_copy(acc.at[pl.ds(sid * rpt, rpt)],
                        accp_hbm.at[cid, pl.ds(sid * rpt, rpt)])

    return scatter_kernel


SI = BS * LP
_edge_kernel = _make_gather_scatter(E, NPAD, K)

SPS = 2
SK = SPS * LP
SIPW = SI // NW
SG = SIPW // SK
VPR = D // 16


@functools.partial(
    pl.kernel,
    out_type=jax.ShapeDtypeStruct((BS, D), jnp.float32),
    mesh=_mesh,
    scratch_types=[
        pltpu.VMEM((SK,), jnp.int32),
        pltpu.VMEM((SK,), jnp.int32),
        pltpu.VMEM((SK, D), jnp.float32),
        pltpu.VMEM((SK, D), jnp.float32),
        pltpu.VMEM((SPW, D), jnp.float32),
        pltpu.SemaphoreType.DMA,
        pltpu.SemaphoreType.DMA,
    ],
)
def _sent_kernel(sidx_hbm, h2_hbm, out_hbm, iA, iB, rA, rB, res, semA, semB):
    wid = _wid()
    base = wid * SIPW
    obase = wid * SPW

    def start(s, ibuf, rbuf, sem):
        pltpu.sync_copy(sidx_hbm.at[pl.ds(base + s * SK, SK)], ibuf)
        pltpu.async_copy(h2_hbm.at[ibuf], rbuf, sem)

    def finish(s, ibuf, rbuf, sem):
        pltpu.make_async_copy(h2_hbm.at[ibuf], rbuf, sem).wait()
        for j in range(SPS):
            def body(r, carry):
                return tuple(carry[c] + rbuf[j * LP + r, pl.ds(c * 16, 16)]
                             for c in range(VPR))
            acc = lax.fori_loop(
                0, 50, body,
                tuple(jnp.zeros((16,), jnp.float32) for _ in range(VPR)))
            for c in range(VPR):
                res[s * SPS + j, pl.ds(c * 16, 16)] = acc[c]

    start(0, iA, rA, semA)

    def body(i, _):
        sB = 2 * i + 1

        @pl.when(sB < SG)
        def _():
            start(sB, iB, rB, semB)

        finish(2 * i, iA, rA, semA)

        @pl.when(2 * i + 2 < SG)
        def _():
            start(2 * i + 2, iA, rA, semA)

        @pl.when(sB < SG)
        def _():
            finish(sB, iB, rB, semB)

        return 0

    lax.fori_loop(0, (SG + 1) // 2, body, 0)
    pltpu.sync_copy(res, out_hbm.at[pl.ds(obase, SPW)])


def _dinv_block(degp, i):
    deg = 1.0 + jnp.sum(degp, axis=0)[:, :1]
    dinv = lax.rsqrt(deg)
    rid = i * BR + lax.broadcasted_iota(jnp.int32, (BR, 1), 0)
    return jnp.where(rid < N, dinv, 0.0)


def _tc1_body(degp_ref, x_ref, u_ref):
    dinv = _dinv_block(degp_ref[...], pl.program_id(0))
    u_ref[...] = x_ref[...] * dinv


def _tc2_body(degp_ref, accp_ref, u1_ref, w1_ref, b1_ref, w2_ref, u2_ref):
    dinv = _dinv_block(degp_ref[...], pl.program_id(0))
    px = (accp_ref[0] + accp_ref[1] + u1_ref[...]) * dinv
    h1 = jax.nn.relu(
        jnp.dot(px, w1_ref[...], preferred_element_type=jnp.float32)
        + b1_ref[...])
    g = jnp.dot(h1, w2_ref[...], preferred_element_type=jnp.float32)
    u2_ref[...] = g * dinv


def _tc3_body(degp_ref, accp_ref, u2_ref, h2_ref):
    dinv = _dinv_block(degp_ref[...], pl.program_id(0))
    h2_ref[...] = (accp_ref[0] + accp_ref[1] + u2_ref[...]) * dinv


def _tc4_body(sent_ref, b2_ref, wf1_ref, bf1_ref, wf2_ref, bf2_ref,
              wf3_ref, bf3_ref, out_ref):
    s = sent_ref[...] + 50.0 * b2_ref[...]
    o1 = jax.nn.relu(
        jnp.dot(s, wf1_ref[...], preferred_element_type=jnp.float32)
        + bf1_ref[...])
    o2 = jax.nn.relu(
        jnp.dot(o1, wf2_ref[...], preferred_element_type=jnp.float32)
        + bf2_ref[...])
    out_ref[...] = (
        jnp.dot(o2, wf3_ref[...], preferred_element_type=jnp.float32)
        + bf3_ref[...])


def _row_grid_call(body, in_specs_extra, out_width, *args):
    grid = (NPAD // BR,)
    degp_spec = pl.BlockSpec((NC, BR, D), lambda i: (0, i, 0))
    return pl.pallas_call(
        body,
        grid=grid,
        in_specs=[degp_spec] + in_specs_extra,
        out_specs=pl.BlockSpec((BR, out_width), lambda i: (i, 0)),
        out_shape=jax.ShapeDtypeStruct((NPAD, out_width), jnp.float32),
    )(*args)


_ROWS = pl.BlockSpec((BR, D), lambda i: (i, 0))
_ACCP = pl.BlockSpec((NC, BR, D), lambda i: (0, i, 0))


def _full(shape):
    return pl.BlockSpec(shape, lambda i: tuple(0 for _ in shape))


def kernel(sentence, x, edge_index, W1, b1, W2, b2,
           Wf1, bf1, Wf2, bf2, Wf3, bf3):
    f32 = jnp.float32
    src = edge_index[0]
    dst = edge_index[1]
    xp = jnp.zeros((NPAD, D), f32).at[:N].set(x)

    degp = _deg_kernel(dst)
    u1 = _row_grid_call(_tc1_body, [_ROWS], D, degp, xp)
    accp1 = _edge_kernel(src, dst, u1)
    u2 = _row_grid_call(
        _tc2_body,
        [_ACCP, _ROWS, _full((D, 256)), _full((1, 256)), _full((256, D))],
        D, degp, accp1, u1, W1, b1.reshape(1, -1), W2)
    accp2 = _edge_kernel(src, dst, u2)
    h2 = _row_grid_call(_tc3_body, [_ACCP, _ROWS], D,
                        degp, accp2, u2)

    sentp = jnp.pad(sentence, ((0, 0), (0, LP - sentence.shape[1])),
                    constant_values=ZERO_ROW).reshape(-1)
    sent = _sent_kernel(sentp, h2)

    wf3p = jnp.zeros((D, 128), f32).at[:, :2].set(Wf3)
    bf3p = jnp.zeros((1, 128), f32).at[0, :2].set(bf3)
    outp = pl.pallas_call(
        _tc4_body,
        grid=(1,),
        in_specs=[_full((BS, D)), _full((1, D)), _full((D, 256)),
                  _full((1, 256)), _full((256, D)), _full((1, D)),
                  _full((D, 128)), _full((1, 128))],
        out_specs=_full((BS, 128)),
        out_shape=jax.ShapeDtypeStruct((BS, 128), f32),
    )(sent, b2.reshape(1, -1), Wf1, bf1.reshape(1, -1), Wf2,
      bf2.reshape(1, -1), wf3p, bf3p)
    return outp[:, :2]


# speedup vs baseline: 1.4077x; 1.3923x over previous
import functools

import jax
import jax.numpy as jnp
from jax import lax
from jax.experimental import pallas as pl
from jax.experimental.pallas import tpu as pltpu
from jax.experimental.pallas import tpu_sc as plsc

N = 10000
NPAD = 10240
E = 320000
D = 128
NC = 2
NS = 16
NW = NC * NS
EPW = E // NW
K = 80
G = EPW // K
RPT = NPAD // NS
BS = 1024
LP = 56
SPW = BS // NW
ZERO_ROW = 10200
BR = 1024

_mesh = plsc.VectorSubcoreMesh(core_axis_name="c", subcore_axis_name="s",
                               num_cores=NC, num_subcores=NS)


def _wid():
    return lax.axis_index("s") * NC + lax.axis_index("c")


def _fill_rows(ref, nrows, width, value):
    vecs = width // 16
    val = jnp.full((16,), value, jnp.float32)

    def body(i, _):
        r = i // vecs
        c = i % vecs
        ref[r, pl.ds(c * 16, 16)] = val
        return 0

    lax.fori_loop(0, nrows * vecs, body, 0)


@functools.partial(
    pl.kernel,
    out_type=jax.ShapeDtypeStruct((NC, NPAD, D), jnp.float32),
    mesh=_mesh,
    scratch_types=[
        pltpu.VMEM((K,), jnp.int32),
        pltpu.VMEM((K, D), jnp.float32),
        pltpu.VMEM_SHARED((NPAD, D), jnp.float32),
    ],
)
def _deg_kernel(dst_hbm, degp_hbm, dbuf, ones_v, acc):
    cid = lax.axis_index("c")
    sid = lax.axis_index("s")
    wid = _wid()
    base = wid * EPW

    _fill_rows(ones_v, K, D, 0.0)
    for j in range(RPT // K):
        pltpu.sync_copy(ones_v, acc.at[pl.ds(sid * RPT + j * K, K)])
    plsc.subcore_barrier()
    _fill_rows(ones_v, K, D, 1.0)

    def body(g, _):
        pltpu.sync_copy(dst_hbm.at[pl.ds(base + g * K, K)], dbuf)
        pltpu.sync_copy(ones_v, acc.at[dbuf], add=True)
        return 0

    lax.fori_loop(0, G, body, 0)
    plsc.subcore_barrier()
    pltpu.sync_copy(acc.at[pl.ds(sid * RPT, RPT)],
                    degp_hbm.at[cid, pl.ds(sid * RPT, RPT)])


def _make_gather_scatter(nidx, nout, k):
    ipw = nidx // NW
    g = ipw // k
    rpt = nout // NS
    nz = min(k, rpt)

    @functools.partial(
        pl.kernel,
        out_type=jax.ShapeDtypeStruct((NC, nout, D), jnp.float32),
        mesh=_mesh,
        scratch_types=[
            pltpu.VMEM((k,), jnp.int32),
            pltpu.VMEM((k,), jnp.int32),
            pltpu.VMEM((k,), jnp.int32),
            pltpu.VMEM((k,), jnp.int32),
            pltpu.VMEM((k, D), jnp.float32),
            pltpu.VMEM((k, D), jnp.float32),
            pltpu.VMEM_SHARED((nout, D), jnp.float32),
            pltpu.SemaphoreType.DMA,
            pltpu.SemaphoreType.DMA,
        ],
    )
    def scatter_kernel(src_hbm, dst_hbm, u_hbm, accp_hbm,
                       sA, sB, dA, dB, rA, rB, acc, semA, semB):
        cid = lax.axis_index("c")
        sid = lax.axis_index("s")
        wid = _wid()
        base = wid * ipw

        _fill_rows(rA, nz, D, 0.0)
        for j in range(rpt // nz):
            pltpu.sync_copy(rA.at[pl.ds(0, nz)],
                            acc.at[pl.ds(sid * rpt + j * nz, nz)])
        plsc.subcore_barrier()

        def start(s, sbuf, rbuf, sem):
            pltpu.sync_copy(src_hbm.at[pl.ds(base + s * k, k)], sbuf)
            pltpu.async_copy(u_hbm.at[sbuf], rbuf, sem)

        def finish(s, sbuf, dbuf, rbuf, sem):
            pltpu.sync_copy(dst_hbm.at[pl.ds(base + s * k, k)], dbuf)
            pltpu.make_async_copy(u_hbm.at[sbuf], rbuf, sem).wait()
            pltpu.sync_copy(rbuf, acc.at[dbuf], add=True)

        start(0, sA, rA, semA)

        def body(i, _):
            gB = 2 * i + 1

            @pl.when(gB < g)
            def _():
                start(gB, sB, rB, semB)

            finish(2 * i, sA, dA, rA, semA)

            @pl.when(2 * i + 2 < g)
            def _():
                start(2 * i + 2, sA, rA, semA)

            @pl.when(gB < g)
            def _():
                finish(gB, sB, dB, rB, semB)

            return 0

        lax.fori_loop(0, (g + 1) // 2, body, 0)
        plsc.subcore_barrier()
        pltpu.sync_copy(acc.at[pl.ds(sid * rpt, rpt)],
                        accp_hbm.at[cid, pl.ds(sid * rpt, rpt)])

    return scatter_kernel


SI = BS * LP
_edge_kernel = _make_gather_scatter(E, NPAD, K)

SPS = 2
SK = SPS * LP
SIPW = SI // NW
SG = SIPW // SK
VPR = D // 16


@functools.partial(
    pl.kernel,
    out_type=jax.ShapeDtypeStruct((BS, D), jnp.float32),
    mesh=_mesh,
    scratch_types=[
        pltpu.VMEM((SK,), jnp.int32),
        pltpu.VMEM((SK,), jnp.int32),
        pltpu.VMEM((SK, D), jnp.float32),
        pltpu.VMEM((SK, D), jnp.float32),
        pltpu.VMEM((SPW, D), jnp.float32),
        pltpu.SemaphoreType.DMA,
        pltpu.SemaphoreType.DMA,
    ],
)
def _sent_kernel(sidx_hbm, h2_hbm, out_hbm, iA, iB, rA, rB, res, semA, semB):
    wid = _wid()
    base = wid * SIPW
    obase = wid * SPW

    def start(s, ibuf, rbuf, sem):
        pltpu.sync_copy(sidx_hbm.at[pl.ds(base + s * SK, SK)], ibuf)
        pltpu.async_copy(h2_hbm.at[ibuf], rbuf, sem)

    def finish(s, ibuf, rbuf, sem):
        pltpu.make_async_copy(h2_hbm.at[ibuf], rbuf, sem).wait()
        for j in range(SPS):
            def body(r, carry):
                return tuple(carry[c] + rbuf[j * LP + r, pl.ds(c * 16, 16)]
                             for c in range(VPR))
            acc = lax.fori_loop(
                0, 50, body,
                tuple(jnp.zeros((16,), jnp.float32) for _ in range(VPR)))
            for c in range(VPR):
                res[s * SPS + j, pl.ds(c * 16, 16)] = acc[c]

    start(0, iA, rA, semA)

    def body(i, _):
        sB = 2 * i + 1

        @pl.when(sB < SG)
        def _():
            start(sB, iB, rB, semB)

        finish(2 * i, iA, rA, semA)

        @pl.when(2 * i + 2 < SG)
        def _():
            start(2 * i + 2, iA, rA, semA)

        @pl.when(sB < SG)
        def _():
            finish(sB, iB, rB, semB)

        return 0

    lax.fori_loop(0, (SG + 1) // 2, body, 0)
    pltpu.sync_copy(res, out_hbm.at[pl.ds(obase, SPW)])


def _dinv_block(degp, i):
    deg = 1.0 + jnp.sum(degp, axis=0)[:, :1]
    dinv = lax.rsqrt(deg)
    rid = i * BR + lax.broadcasted_iota(jnp.int32, (BR, 1), 0)
    return jnp.where(rid < N, dinv, 0.0)


def _tc1_body(degp_ref, x_ref, u_ref):
    dinv = _dinv_block(degp_ref[...], pl.program_id(0))
    u_ref[...] = x_ref[...] * dinv


def _tc2_body(degp_ref, accp_ref, u1_ref, w1_ref, b1_ref, w2_ref, u2_ref):
    dinv = _dinv_block(degp_ref[...], pl.program_id(0))
    px = (accp_ref[0] + accp_ref[1] + u1_ref[...]) * dinv
    h1 = jax.nn.relu(
        jnp.dot(px, w1_ref[...], preferred_element_type=jnp.float32)
        + b1_ref[...])
    g = jnp.dot(h1, w2_ref[...], preferred_element_type=jnp.float32)
    u2_ref[...] = g * dinv


def _tc3_body(degp_ref, accp_ref, u2_ref, h2_ref):
    dinv = _dinv_block(degp_ref[...], pl.program_id(0))
    h2_ref[...] = (accp_ref[0] + accp_ref[1] + u2_ref[...]) * dinv


def _tc4_body(sent_ref, b2_ref, wf1_ref, bf1_ref, wf2_ref, bf2_ref,
              wf3_ref, bf3_ref, out_ref):
    s = sent_ref[...] + 50.0 * b2_ref[...]
    o1 = jax.nn.relu(
        jnp.dot(s, wf1_ref[...], preferred_element_type=jnp.float32)
        + bf1_ref[...])
    o2 = jax.nn.relu(
        jnp.dot(o1, wf2_ref[...], preferred_element_type=jnp.float32)
        + bf2_ref[...])
    out_ref[...] = (
        jnp.dot(o2, wf3_ref[...], preferred_element_type=jnp.float32)
        + bf3_ref[...])


def _row_grid_call(body, in_specs_extra, out_width, *args):
    grid = (NPAD // BR,)
    degp_spec = pl.BlockSpec((NC, BR, D), lambda i: (0, i, 0))
    return pl.pallas_call(
        body,
        grid=grid,
        in_specs=[degp_spec] + in_specs_extra,
        out_specs=pl.BlockSpec((BR, out_width), lambda i: (i, 0)),
        out_shape=jax.ShapeDtypeStruct((NPAD, out_width), jnp.float32),
    )(*args)


_ROWS = pl.BlockSpec((BR, D), lambda i: (i, 0))
_ACCP = pl.BlockSpec((NC, BR, D), lambda i: (0, i, 0))


def _full(shape):
    return pl.BlockSpec(shape, lambda i: tuple(0 for _ in shape))


def kernel(sentence, x, edge_index, W1, b1, W2, b2,
           Wf1, bf1, Wf2, bf2, Wf3, bf3):
    f32 = jnp.float32
    src = edge_index[0]
    dst = edge_index[1]
    xp = jnp.zeros((NPAD, D), f32).at[:N].set(x)

    degp = _deg_kernel(dst)
    u1 = _row_grid_call(_tc1_body, [_ROWS], D, degp, xp)
    accp1 = _edge_kernel(src, dst, u1)
    u2 = _row_grid_call(
        _tc2_body,
        [_ACCP, _ROWS, _full((D, 256)), _full((1, 256)), _full((256, D))],
        D, degp, accp1, u1, W1, b1.reshape(1, -1), W2)
    accp2 = _edge_kernel(src, dst, u2)
    h2 = _row_grid_call(_tc3_body, [_ACCP, _ROWS], D,
                        degp, accp2, u2)

    padv = (N + jnp.arange(SI, dtype=jnp.int32) % (NPAD - N)).reshape(BS, LP)
    real = jnp.pad(sentence.astype(jnp.int32),
                   ((0, 0), (0, LP - sentence.shape[1])))
    sentp = jnp.where(jnp.arange(LP) < sentence.shape[1], real,
                      padv).reshape(-1)
    sent = _sent_kernel(sentp, h2)

    wf3p = jnp.zeros((D, 128), f32).at[:, :2].set(Wf3)
    bf3p = jnp.zeros((1, 128), f32).at[0, :2].set(bf3)
    outp = pl.pallas_call(
        _tc4_body,
        grid=(1,),
        in_specs=[_full((BS, D)), _full((1, D)), _full((D, 256)),
                  _full((1, 256)), _full((256, D)), _full((1, D)),
                  _full((D, 128)), _full((1, 128))],
        out_specs=_full((BS, 128)),
        out_shape=jax.ShapeDtypeStruct((BS, 128), f32),
    )(sent, b2.reshape(1, -1), Wf1, bf1.reshape(1, -1), Wf2,
      bf2.reshape(1, -1), wf3p, bf3p)
    return outp[:, :2]


# speedup vs baseline: 1.7746x; 1.2606x over previous
import functools

import jax
import jax.numpy as jnp
from jax import lax
from jax.experimental import pallas as pl
from jax.experimental.pallas import tpu as pltpu
from jax.experimental.pallas import tpu_sc as plsc

N = 10000
NPAD = 10240
E = 320000
D = 128
NC = 2
NS = 16
NW = NC * NS
EPW = E // NW
K = 80
G = EPW // K
RPT = NPAD // NS
BS = 1024
LP = 56
SPW = BS // NW
ZERO_ROW = 10200
BR = 1024

_mesh = plsc.VectorSubcoreMesh(core_axis_name="c", subcore_axis_name="s",
                               num_cores=NC, num_subcores=NS)


def _wid():
    return lax.axis_index("s") * NC + lax.axis_index("c")


def _fill_rows(ref, nrows, width, value):
    vecs = width // 16
    val = jnp.full((16,), value, jnp.float32)

    def body(i, _):
        r = i // vecs
        c = i % vecs
        ref[r, pl.ds(c * 16, 16)] = val
        return 0

    lax.fori_loop(0, nrows * vecs, body, 0)


DW = 128


@functools.partial(
    pl.kernel,
    out_type=jax.ShapeDtypeStruct((NC, NPAD, DW), jnp.float32),
    mesh=_mesh,
    scratch_types=[
        pltpu.VMEM((G, K), jnp.int32),
        pltpu.VMEM((K, DW), jnp.float32),
        pltpu.VMEM_SHARED((NPAD, DW), jnp.float32),
    ],
)
def _deg_kernel(dst_hbm, degp_hbm, dbig, ones_v, acc):
    cid = lax.axis_index("c")
    sid = lax.axis_index("s")
    wid = _wid()

    _fill_rows(ones_v, K, DW, 0.0)
    for j in range(RPT // K):
        pltpu.sync_copy(ones_v, acc.at[pl.ds(sid * RPT + j * K, K)])
    plsc.subcore_barrier()
    _fill_rows(ones_v, K, DW, 1.0)
    pltpu.sync_copy(dst_hbm.at[wid], dbig)

    def body(g, _):
        pltpu.sync_copy(ones_v, acc.at[dbig.at[g]], add=True)
        return 0

    lax.fori_loop(0, G, body, 0)
    plsc.subcore_barrier()
    pltpu.sync_copy(acc.at[pl.ds(sid * RPT, RPT)],
                    degp_hbm.at[cid, pl.ds(sid * RPT, RPT)])


@functools.partial(
    pl.kernel,
    out_type=jax.ShapeDtypeStruct((NC, NPAD, D), jnp.float32),
    mesh=_mesh,
    scratch_types=[
        pltpu.VMEM((K,), jnp.int32),
        pltpu.VMEM((K,), jnp.int32),
        pltpu.VMEM((G, K), jnp.int32),
        pltpu.VMEM((K, D), jnp.float32),
        pltpu.VMEM((K, D), jnp.float32),
        pltpu.VMEM_SHARED((NPAD, D), jnp.float32),
        pltpu.SemaphoreType.DMA,
        pltpu.SemaphoreType.DMA,
    ],
)
def _edge_kernel(src_hbm, dst_hbm, u_hbm, accp_hbm,
                 sA, sB, dbig, rA, rB, acc, semA, semB):
    cid = lax.axis_index("c")
    sid = lax.axis_index("s")
    wid = _wid()

    _fill_rows(rA, K, D, 0.0)
    for j in range(RPT // K):
        pltpu.sync_copy(rA.at[pl.ds(0, K)],
                        acc.at[pl.ds(sid * RPT + j * K, K)])
    plsc.subcore_barrier()
    pltpu.sync_copy(dst_hbm.at[wid], dbig)

    def start(s, sbuf, rbuf, sem):
        pltpu.sync_copy(src_hbm.at[wid, s], sbuf)
        pltpu.async_copy(u_hbm.at[sbuf], rbuf, sem)

    def finish(s, sbuf, rbuf, sem):
        pltpu.make_async_copy(u_hbm.at[sbuf], rbuf, sem).wait()
        pltpu.sync_copy(rbuf, acc.at[dbig.at[s]], add=True)

    start(0, sA, rA, semA)

    def body(i, _):
        gB = 2 * i + 1

        @pl.when(gB < G)
        def _():
            start(gB, sB, rB, semB)

        finish(2 * i, sA, rA, semA)

        @pl.when(2 * i + 2 < G)
        def _():
            start(2 * i + 2, sA, rA, semA)

        @pl.when(gB < G)
        def _():
            finish(gB, sB, rB, semB)

        return 0

    lax.fori_loop(0, (G + 1) // 2, body, 0)
    plsc.subcore_barrier()
    pltpu.sync_copy(acc.at[pl.ds(sid * RPT, RPT)],
                    accp_hbm.at[cid, pl.ds(sid * RPT, RPT)])


SI = BS * LP

SPS = 2
SK = SPS * LP
SIPW = SI // NW
SG = SIPW // SK
VPR = D // 16


@functools.partial(
    pl.kernel,
    out_type=jax.ShapeDtypeStruct((BS, D), jnp.float32),
    mesh=_mesh,
    scratch_types=[
        pltpu.VMEM((SG, SK), jnp.int32),
        pltpu.VMEM((SK, D), jnp.float32),
        pltpu.VMEM((SK, D), jnp.float32),
        pltpu.VMEM((SPW, D), jnp.float32),
        pltpu.SemaphoreType.DMA,
        pltpu.SemaphoreType.DMA,
    ],
)
def _sent_kernel(sidx_hbm, h2_hbm, out_hbm, ibig, rA, rB, res, semA, semB):
    wid = _wid()
    obase = wid * SPW
    pltpu.sync_copy(sidx_hbm.at[wid], ibig)

    def start(s, rbuf, sem):
        pltpu.async_copy(h2_hbm.at[ibig.at[s]], rbuf, sem)

    def finish(s, rbuf, sem):
        pltpu.make_async_copy(h2_hbm.at[ibig.at[s]], rbuf, sem).wait()
        for j in range(SPS):
            def body(r, carry):
                return tuple(carry[c] + rbuf[j * LP + r, pl.ds(c * 16, 16)]
                             for c in range(VPR))
            acc = lax.fori_loop(
                0, 50, body,
                tuple(jnp.zeros((16,), jnp.float32) for _ in range(VPR)))
            for c in range(VPR):
                res[s * SPS + j, pl.ds(c * 16, 16)] = acc[c]

    start(0, rA, semA)

    def body(i, _):
        sB = 2 * i + 1

        @pl.when(sB < SG)
        def _():
            start(sB, rB, semB)

        finish(2 * i, rA, semA)

        @pl.when(2 * i + 2 < SG)
        def _():
            start(2 * i + 2, rA, semA)

        @pl.when(sB < SG)
        def _():
            finish(sB, rB, semB)

        return 0

    lax.fori_loop(0, (SG + 1) // 2, body, 0)
    pltpu.sync_copy(res, out_hbm.at[pl.ds(obase, SPW)])


def _dinv_block(degp, i):
    deg = 1.0 + jnp.sum(degp, axis=0)[:, :1]
    dinv = lax.rsqrt(deg)
    rid = i * BR + lax.broadcasted_iota(jnp.int32, (BR, 1), 0)
    return jnp.where(rid < N, dinv, 0.0)


def _tc1_body(degp_ref, x_ref, u_ref):
    dinv = _dinv_block(degp_ref[...], pl.program_id(0))
    u_ref[...] = x_ref[...] * dinv


def _tc2_body(degp_ref, accp_ref, u1_ref, w1_ref, b1_ref, w2_ref, u2_ref):
    dinv = _dinv_block(degp_ref[...], pl.program_id(0))
    px = (accp_ref[0] + accp_ref[1] + u1_ref[...]) * dinv
    h1 = jax.nn.relu(
        jnp.dot(px, w1_ref[...], preferred_element_type=jnp.float32)
        + b1_ref[...])
    g = jnp.dot(h1, w2_ref[...], preferred_element_type=jnp.float32)
    u2_ref[...] = g * dinv


def _tc3_body(degp_ref, accp_ref, u2_ref, h2_ref):
    dinv = _dinv_block(degp_ref[...], pl.program_id(0))
    h2_ref[...] = (accp_ref[0] + accp_ref[1] + u2_ref[...]) * dinv


def _tc4_body(sent_ref, b2_ref, wf1_ref, bf1_ref, wf2_ref, bf2_ref,
              wf3_ref, bf3_ref, out_ref):
    s = sent_ref[...] + 50.0 * b2_ref[...]
    o1 = jax.nn.relu(
        jnp.dot(s, wf1_ref[...], preferred_element_type=jnp.float32)
        + bf1_ref[...])
    o2 = jax.nn.relu(
        jnp.dot(o1, wf2_ref[...], preferred_element_type=jnp.float32)
        + bf2_ref[...])
    out_ref[...] = (
        jnp.dot(o2, wf3_ref[...], preferred_element_type=jnp.float32)
        + bf3_ref[...])


def _row_grid_call(body, in_specs_extra, out_width, *args):
    grid = (NPAD // BR,)
    degp_spec = pl.BlockSpec((NC, BR, DW), lambda i: (0, i, 0))
    return pl.pallas_call(
        body,
        grid=grid,
        in_specs=[degp_spec] + in_specs_extra,
        out_specs=pl.BlockSpec((BR, out_width), lambda i: (i, 0)),
        out_shape=jax.ShapeDtypeStruct((NPAD, out_width), jnp.float32),
    )(*args)


_ROWS = pl.BlockSpec((BR, D), lambda i: (i, 0))
_ACCP = pl.BlockSpec((NC, BR, D), lambda i: (0, i, 0))


def _full(shape):
    return pl.BlockSpec(shape, lambda i: tuple(0 for _ in shape))


def kernel(sentence, x, edge_index, W1, b1, W2, b2,
           Wf1, bf1, Wf2, bf2, Wf3, bf3):
    f32 = jnp.float32
    src = edge_index[0].reshape(NW, G, K)
    dst = edge_index[1].reshape(NW, G, K)
    xp = jnp.zeros((NPAD, D), f32).at[:N].set(x)

    degp = _deg_kernel(dst)
    u1 = _row_grid_call(_tc1_body, [_ROWS], D, degp, xp)
    accp1 = _edge_kernel(src, dst, u1)
    u2 = _row_grid_call(
        _tc2_body,
        [_ACCP, _ROWS, _full((D, 256)), _full((1, 256)), _full((256, D))],
        D, degp, accp1, u1, W1, b1.reshape(1, -1), W2)
    accp2 = _edge_kernel(src, dst, u2)
    h2 = _row_grid_call(_tc3_body, [_ACCP, _ROWS], D,
                        degp, accp2, u2)

    padv = (N + jnp.arange(SI, dtype=jnp.int32) % (NPAD - N)).reshape(BS, LP)
    real = jnp.pad(sentence.astype(jnp.int32),
                   ((0, 0), (0, LP - sentence.shape[1])))
    sentp = jnp.where(jnp.arange(LP) < sentence.shape[1], real,
                      padv).reshape(NW, SG, SK)
    sent = _sent_kernel(sentp, h2)

    wf3p = jnp.zeros((D, 128), f32).at[:, :2].set(Wf3)
    bf3p = jnp.zeros((1, 128), f32).at[0, :2].set(bf3)
    outp = pl.pallas_call(
        _tc4_body,
        grid=(1,),
        in_specs=[_full((BS, D)), _full((1, D)), _full((D, 256)),
                  _full((1, 256)), _full((256, D)), _full((1, D)),
                  _full((D, 128)), _full((1, 128))],
        out_specs=_full((BS, 128)),
        out_shape=jax.ShapeDtypeStruct((BS, 128), f32),
    )(sent, b2.reshape(1, -1), Wf1, bf1.reshape(1, -1), Wf2,
      bf2.reshape(1, -1), wf3p, bf3p)
    return outp[:, :2]
